# baseline (speedup 1.0000x reference)
import functools

import jax
import jax.numpy as jnp
from jax import lax
from jax.experimental import pallas as pl
from jax.experimental.pallas import tpu as pltpu
from jax.experimental.pallas import tpu_sc as plsc

B, C, Hs, Ws = 32, 1024, 32, 32
E = 64
TOP_K = 2
HW = Hs * Ws
NROWS = B * C
NW = 32
ROWS_W = NROWS // NW
CH = 16
NBUF = 2
NCH = ROWS_W // CH

_mesh = plsc.VectorSubcoreMesh(core_axis_name="c", subcore_axis_name="s")


@functools.partial(
    pl.kernel,
    out_type=jax.ShapeDtypeStruct((NROWS * 16,), jnp.float32),
    mesh=_mesh,
    scratch_types=[
        pltpu.VMEM((NBUF, CH, HW), jnp.float32),
        pltpu.VMEM((ROWS_W * 16,), jnp.float32),
        pltpu.SemaphoreType.DMA((NBUF,)),
    ],
)
def _sc_rowsum(mh_hbm, x_hbm, buf, xout, sems):
    wid = lax.axis_index("s") * 2 + lax.axis_index("c")
    base = wid * ROWS_W

    def chunk_src(ch):
        return mh_hbm.at[pl.ds(base + ch * CH, CH)]

    for s in range(NBUF):
        pltpu.async_copy(chunk_src(s), buf.at[s], sems.at[s])

    @pl.loop(0, NCH, step=NBUF)
    def _chunks(ch0):
        for s in range(NBUF):
            ch = ch0 + s
            pltpu.make_async_copy(chunk_src(ch), buf.at[s], sems.at[s]).wait()
            for r in range(CH):
                acc = buf[s, r, pl.ds(0, 16)]
                for j in range(1, HW // 16):
                    acc = acc + buf[s, r, pl.ds(j * 16, 16)]
                xout[pl.ds(pl.multiple_of((ch * CH + r) * 16, 16), 16)] = acc

            nxt = ch + NBUF

            @pl.when(nxt < NCH)
            def _prefetch():
                pltpu.async_copy(chunk_src(nxt), buf.at[s], sems.at[s])

    pltpu.sync_copy(xout, x_hbm.at[pl.ds(base * 16, ROWS_W * 16)])


def _tc_epilogue(x_ref, noise_ref, wr_ref, br_ref, wn_ref, bn_ref,
                 router_ref, idx_ref, noisy_ref):
    sums = jnp.sum(x_ref[...], axis=2)
    inv_hw = jnp.float32(1.0 / HW)
    dims = (((1,), (1,)), ((), ()))
    route_logits = jax.lax.dot_general(
        sums, wr_ref[...], dims, preferred_element_type=jnp.float32,
        precision=jax.lax.Precision.HIGHEST) * inv_hw + br_ref[...]
    noise_logits = jax.lax.dot_general(
        sums, wn_ref[...], dims, preferred_element_type=jnp.float32,
        precision=jax.lax.Precision.HIGHEST) * inv_hw + bn_ref[...]

    def softmax(v):
        m = jnp.max(v, axis=1, keepdims=True)
        e = jnp.exp(v - m)
        return e / jnp.sum(e, axis=1, keepdims=True)

    logits = softmax(route_logits)
    n = softmax(noise_ref[...] * jax.nn.softplus(noise_logits))
    noisy = logits + n
    noisy_ref[...] = noisy

    iota = jax.lax.broadcasted_iota(jnp.int32, (B, E), 1)
    big = jnp.int32(E)
    v1 = jnp.max(noisy, axis=1, keepdims=True)
    i1 = jnp.min(jnp.where(noisy == v1, iota, big), axis=1, keepdims=True)
    masked = jnp.where(iota == i1, -jnp.inf, noisy)
    v2 = jnp.max(masked, axis=1, keepdims=True)
    i2 = jnp.min(jnp.where(masked == v2, iota, big), axis=1, keepdims=True)

    iota2 = jax.lax.broadcasted_iota(jnp.int32, (B, TOP_K), 1)
    idx_ref[...] = jnp.where(iota2 == 0, i1, i2)
    e2 = jnp.exp(v2 - v1)
    denom = 1.0 + e2
    router_ref[...] = jnp.where(iota2 == 0, 1.0 / denom, e2 / denom)


@jax.jit
def kernel(mh_output, noise, W_route, b_route, W_noise, b_noise):
    mh = mh_output.reshape(NROWS, HW)
    x = _sc_rowsum(mh).reshape(B, C, 16)
    br = b_route.reshape(1, E)
    bn = b_noise.reshape(1, E)
    router_output, indices, noisy_logits = pl.pallas_call(
        _tc_epilogue,
        out_shape=[
            jax.ShapeDtypeStruct((B, TOP_K), jnp.float32),
            jax.ShapeDtypeStruct((B, TOP_K), jnp.int32),
            jax.ShapeDtypeStruct((B, E), jnp.float32),
        ],
    )(x, noise, W_route, br, W_noise, bn)
    return (router_output, indices, noisy_logits)


# speedup vs baseline: 1.2119x; 1.2119x over previous
import functools

import jax
import jax.numpy as jnp
from jax import lax
from jax.experimental import pallas as pl
from jax.experimental.pallas import tpu as pltpu
from jax.experimental.pallas import tpu_sc as plsc

B, C, Hs, Ws = 32, 1024, 32, 32
E = 64
TOP_K = 2
HW = Hs * Ws
NROWS = B * C
NW = 32
ROWS_W = NROWS // NW
CH = 16
NBUF = 2
NCH = ROWS_W // CH

_mesh = plsc.VectorSubcoreMesh(core_axis_name="c", subcore_axis_name="s")


@functools.partial(
    pl.kernel,
    out_type=jax.ShapeDtypeStruct((NROWS * 16,), jnp.float32),
    mesh=_mesh,
    scratch_types=[
        pltpu.VMEM((NBUF, CH, HW), jnp.float32),
        pltpu.VMEM((ROWS_W * 16,), jnp.float32),
        pltpu.SemaphoreType.DMA((NBUF,)),
    ],
)
def _sc_rowsum(mh_hbm, x_hbm, buf, xout, sems):
    wid = lax.axis_index("s") * 2 + lax.axis_index("c")
    base = wid * ROWS_W

    def chunk_src(ch):
        return mh_hbm.at[pl.ds(base + ch * CH, CH)]

    for s in range(NBUF):
        pltpu.async_copy(chunk_src(s), buf.at[s], sems.at[s])

    @pl.loop(0, NCH, step=NBUF)
    def _chunks(ch0):
        for s in range(NBUF):
            ch = ch0 + s
            pltpu.make_async_copy(chunk_src(ch), buf.at[s], sems.at[s]).wait()
            for r in range(CH):
                accs = [buf[s, r, pl.ds(k * 16, 16)] for k in range(8)]
                for j in range(8, HW // 16):
                    accs[j % 8] = accs[j % 8] + buf[s, r, pl.ds(j * 16, 16)]
                acc = ((accs[0] + accs[1]) + (accs[2] + accs[3])) + (
                    (accs[4] + accs[5]) + (accs[6] + accs[7]))
                xout[pl.ds(pl.multiple_of((ch * CH + r) * 16, 16), 16)] = acc

            nxt = ch + NBUF

            @pl.when(nxt < NCH)
            def _prefetch():
                pltpu.async_copy(chunk_src(nxt), buf.at[s], sems.at[s])

    pltpu.sync_copy(xout, x_hbm.at[pl.ds(base * 16, ROWS_W * 16)])


def _tc_epilogue(x_ref, noise_ref, wr_ref, br_ref, wn_ref, bn_ref,
                 router_ref, idx_ref, noisy_ref):
    sums = jnp.sum(x_ref[...], axis=2)
    inv_hw = jnp.float32(1.0 / HW)
    dims = (((1,), (1,)), ((), ()))
    route_logits = jax.lax.dot_general(
        sums, wr_ref[...], dims, preferred_element_type=jnp.float32,
        precision=jax.lax.Precision.HIGHEST) * inv_hw + br_ref[...]
    noise_logits = jax.lax.dot_general(
        sums, wn_ref[...], dims, preferred_element_type=jnp.float32,
        precision=jax.lax.Precision.HIGHEST) * inv_hw + bn_ref[...]

    def softmax(v):
        m = jnp.max(v, axis=1, keepdims=True)
        e = jnp.exp(v - m)
        return e / jnp.sum(e, axis=1, keepdims=True)

    logits = softmax(route_logits)
    n = softmax(noise_ref[...] * jax.nn.softplus(noise_logits))
    noisy = logits + n
    noisy_ref[...] = noisy

    iota = jax.lax.broadcasted_iota(jnp.int32, (B, E), 1)
    big = jnp.int32(E)
    v1 = jnp.max(noisy, axis=1, keepdims=True)
    i1 = jnp.min(jnp.where(noisy == v1, iota, big), axis=1, keepdims=True)
    masked = jnp.where(iota == i1, -jnp.inf, noisy)
    v2 = jnp.max(masked, axis=1, keepdims=True)
    i2 = jnp.min(jnp.where(masked == v2, iota, big), axis=1, keepdims=True)

    iota2 = jax.lax.broadcasted_iota(jnp.int32, (B, TOP_K), 1)
    idx_ref[...] = jnp.where(iota2 == 0, i1, i2)
    e2 = jnp.exp(v2 - v1)
    denom = 1.0 + e2
    router_ref[...] = jnp.where(iota2 == 0, 1.0 / denom, e2 / denom)


@jax.jit
def kernel(mh_output, noise, W_route, b_route, W_noise, b_noise):
    mh = mh_output.reshape(NROWS, HW)
    x = _sc_rowsum(mh).reshape(B, C, 16)
    br = b_route.reshape(1, E)
    bn = b_noise.reshape(1, E)
    router_output, indices, noisy_logits = pl.pallas_call(
        _tc_epilogue,
        out_shape=[
            jax.ShapeDtypeStruct((B, TOP_K), jnp.float32),
            jax.ShapeDtypeStruct((B, TOP_K), jnp.int32),
            jax.ShapeDtypeStruct((B, E), jnp.float32),
        ],
    )(x, noise, W_route, br, W_noise, bn)
    return (router_output, indices, noisy_logits)
